# Initial kernel scaffold; baseline (speedup 1.0000x reference)
import jax
import jax.numpy as jnp
from jax.experimental import pallas as pl


def kernel(x, edge_index, edge_pairs, edge_attr, Wl1, bl1, Wr1, Wl2, bl2, Wr2, W1, b1, W2, b2):
    raise NotImplementedError("write your pallas kernel here")


# speedup vs baseline: 4.1709x; 4.1709x over previous
import functools

import jax
import jax.numpy as jnp
from jax import lax
from jax.experimental import pallas as pl
from jax.experimental.pallas import tpu as pltpu
from jax.experimental.pallas import tpu_sc as plsc

N = 10000
E = 320000
P = 100000
D = 128
H = 128

NC = 2
NS = 16
NW = NC * NS
CH = 128

STRIPE = 640
LAST = N - (NS - 1) * STRIPE
ZR = 8
NCH_E = E // CH
NCH_P = P // CH + 1
EK = (NCH_E + NW - 1) // NW
PK = (NCH_P + NW - 1) // NW

_mesh = plsc.VectorSubcoreMesh(core_axis_name="c", subcore_axis_name="s")


def _zero_vmem_2d(ref, nrows):
    def body(i, _):
        ref[i // 8, pl.ds((i % 8) * 16, 16)] = jnp.zeros((16,), jnp.float32)
        return 0
    lax.fori_loop(0, nrows * 8, body, 0)


def _seg_body(y_hbm, src_hbm, dst_hbm, pout,
              sidx, didx, rows, zbuf, sem, table_sh):
    cid = lax.axis_index("c")
    sid = lax.axis_index("s")
    wid = sid * NC + cid

    _zero_vmem_2d(zbuf, ZR)
    r0 = sid * STRIPE

    @pl.when(sid < NS - 1)
    def _():
        for r in range(STRIPE // ZR):
            pltpu.sync_copy(zbuf, table_sh.at[pl.ds(r0 + r * ZR, ZR)])

    @pl.when(sid == NS - 1)
    def _():
        for r in range(LAST // ZR):
            pltpu.sync_copy(zbuf, table_sh.at[pl.ds(r0 + r * ZR, ZR)])
    plsc.subcore_barrier()

    def chunk(k, _):
        c = wid + k * NW

        @pl.when(c < NCH_E)
        def _():
            base = c * CH
            pltpu.sync_copy(src_hbm.at[pl.ds(base, CH)], sidx.at[0])
            pltpu.sync_copy(dst_hbm.at[pl.ds(base, CH)], didx.at[0])
            pltpu.async_copy(y_hbm.at[sidx.at[0]], rows, sem).wait()
            pltpu.sync_copy(rows, table_sh.at[didx.at[0]], add=True)
        return 0

    lax.fori_loop(0, EK, chunk, 0)
    plsc.subcore_barrier()

    @pl.when(sid < NS - 1)
    def _():
        pltpu.sync_copy(table_sh.at[pl.ds(r0, STRIPE)],
                        pout.at[pl.ds(cid * N + r0, STRIPE)])

    @pl.when(sid == NS - 1)
    def _():
        pltpu.sync_copy(table_sh.at[pl.ds(r0, LAST)],
                        pout.at[pl.ds(cid * N + r0, LAST)])


_segment_sum = functools.partial(
    pl.kernel, _seg_body,
    out_type=jax.ShapeDtypeStruct((NC * N, H), jnp.float32),
    mesh=_mesh,
    scratch_types=[
        pltpu.VMEM((1, CH), jnp.int32),
        pltpu.VMEM((1, CH), jnp.int32),
        pltpu.VMEM((CH, H), jnp.float32),
        pltpu.VMEM((ZR, H), jnp.float32),
        pltpu.SemaphoreType.DMA,
        pltpu.VMEM_SHARED((N, H), jnp.float32),
    ],
)()


def _cnt_body(dst_hbm, ones_hbm, cntout, didx, ones_v, zbuf, cnt_sh):
    cid = lax.axis_index("c")
    sid = lax.axis_index("s")
    wid = sid * NC + cid

    pltpu.sync_copy(ones_hbm, ones_v)
    _zero_vmem_2d(zbuf, ZR)
    r0 = sid * STRIPE

    @pl.when(sid < NS - 1)
    def _():
        for r in range(STRIPE // ZR):
            pltpu.sync_copy(zbuf, cnt_sh.at[pl.ds(r0 + r * ZR, ZR)])

    @pl.when(sid == NS - 1)
    def _():
        for r in range(LAST // ZR):
            pltpu.sync_copy(zbuf, cnt_sh.at[pl.ds(r0 + r * ZR, ZR)])
    plsc.subcore_barrier()

    def chunk(k, _):
        c = wid + k * NW

        @pl.when(c < NCH_E)
        def _():
            base = c * CH
            pltpu.sync_copy(dst_hbm.at[pl.ds(base, CH)], didx.at[0])
            pltpu.sync_copy(ones_v, cnt_sh.at[didx.at[0]], add=True)
        return 0

    lax.fori_loop(0, EK, chunk, 0)
    plsc.subcore_barrier()

    @pl.when(sid < NS - 1)
    def _():
        pltpu.sync_copy(cnt_sh.at[pl.ds(r0, STRIPE)],
                        cntout.at[pl.ds(cid * N + r0, STRIPE)])

    @pl.when(sid == NS - 1)
    def _():
        pltpu.sync_copy(cnt_sh.at[pl.ds(r0, LAST)],
                        cntout.at[pl.ds(cid * N + r0, LAST)])


_count_dst = functools.partial(
    pl.kernel, _cnt_body,
    out_type=jax.ShapeDtypeStruct((NC * N, H), jnp.float32),
    mesh=_mesh,
    scratch_types=[
        pltpu.VMEM((1, CH), jnp.int32),
        pltpu.VMEM((CH, H), jnp.float32),
        pltpu.VMEM((ZR, H), jnp.float32),
        pltpu.VMEM_SHARED((N, H), jnp.float32),
    ],
)()


def _pair_body(zu_hbm, zv_hbm, u_hbm, v_hbm, g1out, g2out,
               uidx, vidx, bufa, sem):
    cid = lax.axis_index("c")
    sid = lax.axis_index("s")
    wid = sid * NC + cid

    def chunk(k, _):
        c = wid + k * NW

        @pl.when(c < NCH_P)
        def _():
            base = jnp.minimum(c * CH, P - CH)
            pltpu.sync_copy(u_hbm.at[pl.ds(base, CH)], uidx.at[0])
            pltpu.sync_copy(v_hbm.at[pl.ds(base, CH)], vidx.at[0])
            pltpu.async_copy(zu_hbm.at[uidx.at[0]], bufa, sem).wait()
            pltpu.sync_copy(bufa, g1out.at[pl.ds(base, CH)])
            pltpu.async_copy(zv_hbm.at[vidx.at[0]], bufa, sem).wait()
            pltpu.sync_copy(bufa, g2out.at[pl.ds(base, CH)])
        return 0

    lax.fori_loop(0, PK, chunk, 0)


_pair_gather = functools.partial(
    pl.kernel, _pair_body,
    out_type=[jax.ShapeDtypeStruct((P, H), jnp.float32),
              jax.ShapeDtypeStruct((P, H), jnp.float32)],
    mesh=_mesh,
    scratch_types=[
        pltpu.VMEM((1, CH), jnp.int32),
        pltpu.VMEM((1, CH), jnp.int32),
        pltpu.VMEM((CH, H), jnp.float32),
        pltpu.SemaphoreType.DMA,
    ],
)()


BN = 1000
BP = 1000


def _pre_body(x_ref, wa_ref, wb_ref, ya_ref, yb_ref):
    xb = x_ref[...]
    ya_ref[...] = jnp.dot(xb, wa_ref[...], preferred_element_type=jnp.float32)
    yb_ref[...] = jnp.dot(xb, wb_ref[...], preferred_element_type=jnp.float32)


_pre_transform = pl.pallas_call(
    _pre_body,
    grid=(N // BN,),
    in_specs=[
        pl.BlockSpec((BN, D), lambda i: (i, 0)),
        pl.BlockSpec((D, H), lambda i: (0, 0)),
        pl.BlockSpec((D, H), lambda i: (0, 0)),
    ],
    out_specs=[
        pl.BlockSpec((BN, H), lambda i: (i, 0)),
        pl.BlockSpec((BN, H), lambda i: (i, 0)),
    ],
    out_shape=[jax.ShapeDtypeStruct((N, H), jnp.float32),
               jax.ShapeDtypeStruct((N, H), jnp.float32)],
)


def _comb_body(p0_ref, p1_ref, c0_ref, c1_ref, yr_ref, bl_ref,
               wa_ref, wb_ref, oa_ref, ob_ref):
    cnt = c0_ref[:, :1] + c1_ref[:, :1]
    recip = 1.0 / jnp.maximum(cnt, 1.0)
    z = (p0_ref[...] + p1_ref[...]) * recip + bl_ref[...] + yr_ref[...]
    z = jnp.maximum(z, 0.0)
    oa_ref[...] = jnp.dot(z, wa_ref[...], preferred_element_type=jnp.float32)
    ob_ref[...] = jnp.dot(z, wb_ref[...], preferred_element_type=jnp.float32)


_combine = pl.pallas_call(
    _comb_body,
    grid=(N // BN,),
    in_specs=[
        pl.BlockSpec((BN, H), lambda i: (i, 0)),
        pl.BlockSpec((BN, H), lambda i: (N // BN + i, 0)),
        pl.BlockSpec((BN, H), lambda i: (i, 0)),
        pl.BlockSpec((BN, H), lambda i: (N // BN + i, 0)),
        pl.BlockSpec((BN, H), lambda i: (i, 0)),
        pl.BlockSpec((1, H), lambda i: (0, 0)),
        pl.BlockSpec((H, H), lambda i: (0, 0)),
        pl.BlockSpec((H, H), lambda i: (0, 0)),
    ],
    out_specs=[
        pl.BlockSpec((BN, H), lambda i: (i, 0)),
        pl.BlockSpec((BN, H), lambda i: (i, 0)),
    ],
    out_shape=[jax.ShapeDtypeStruct((N, H), jnp.float32),
               jax.ShapeDtypeStruct((N, H), jnp.float32)],
)


def _dec_body(g1_ref, g2_ref, ea_ref, w1e_ref, b1_ref, w2_ref, b2_ref, out_ref):
    t = (g1_ref[...] + g2_ref[...] + b1_ref[...]
         + jnp.dot(ea_ref[...], w1e_ref[...], preferred_element_type=jnp.float32))
    t = jnp.maximum(t, 0.0)
    out_ref[0, 0, :] = jnp.sum(t * w2_ref[...], axis=1) + b2_ref[0, 0]


_decode = pl.pallas_call(
    _dec_body,
    grid=(P // BP,),
    in_specs=[
        pl.BlockSpec((BP, H), lambda i: (i, 0)),
        pl.BlockSpec((BP, H), lambda i: (i, 0)),
        pl.BlockSpec((BP, D), lambda i: (i, 0)),
        pl.BlockSpec((D, H), lambda i: (0, 0)),
        pl.BlockSpec((1, H), lambda i: (0, 0)),
        pl.BlockSpec((1, H), lambda i: (0, 0)),
        pl.BlockSpec((1, 1), lambda i: (0, 0)),
    ],
    out_specs=pl.BlockSpec((1, 1, BP), lambda i: (i, 0, 0)),
    out_shape=jax.ShapeDtypeStruct((P // BP, 1, BP), jnp.float32),
)


def kernel(x, edge_index, edge_pairs, edge_attr, Wl1, bl1, Wr1,
           Wl2, bl2, Wr2, W1, b1, W2, b2):
    src = edge_index[0]
    dst = edge_index[1]
    u = edge_pairs[0]
    v = edge_pairs[1]

    W1uT = W1[:, :H].T
    W1vT = W1[:, H:2 * H].T
    W1eT = W1[:, 2 * H:].T
    b1r = b1.reshape(1, H)
    w2r = W2.reshape(1, H)
    b2r = b2.reshape(1, 1)

    yl1, yr1 = _pre_transform(x, Wl1.T, Wr1.T)
    cnt2 = _count_dst(dst, jnp.ones((CH, H), jnp.float32))
    p1 = _segment_sum(yl1, src, dst)
    yl2, yr2 = _combine(p1, p1, cnt2, cnt2, yr1, bl1.reshape(1, H),
                        Wl2.T, Wr2.T)
    p2 = _segment_sum(yl2, src, dst)
    zu, zv = _combine(p2, p2, cnt2, cnt2, yr2, bl2.reshape(1, H),
                      W1uT, W1vT)
    g1, g2 = _pair_gather(zu, zv, u, v)
    out = _decode(g1, g2, edge_attr, W1eT, b1r, w2r, b2r)
    return out.reshape(P)


# speedup vs baseline: 5.2745x; 1.2646x over previous
import functools

import jax
import jax.numpy as jnp
from jax import lax
from jax.experimental import pallas as pl
from jax.experimental.pallas import tpu as pltpu
from jax.experimental.pallas import tpu_sc as plsc

N = 10000
E = 320000
P = 100000
D = 128
H = 128

NC = 2
NS = 16
NW = NC * NS
CH = 128

STRIPE = 640
LAST = N - (NS - 1) * STRIPE
ZR = 8
NCH_E = E // CH
NCH_P = P // CH + 1
EK = (NCH_E + NW - 1) // NW
PK = (NCH_P + NW - 1) // NW

_mesh = plsc.VectorSubcoreMesh(core_axis_name="c", subcore_axis_name="s")


def _zero_vmem_2d(ref, nrows):
    def body(i, _):
        ref[i // 8, pl.ds((i % 8) * 16, 16)] = jnp.zeros((16,), jnp.float32)
        return 0
    lax.fori_loop(0, nrows * 8, body, 0)


def _seg_body(y_hbm, src_hbm, dst_hbm, pout,
              sidx0, sidx1, didx, rows0, rows1, zbuf, sem0, sem1, table_sh):
    cid = lax.axis_index("c")
    sid = lax.axis_index("s")
    wid = sid * NC + cid
    sbufs = (sidx0, sidx1)
    rbufs = (rows0, rows1)
    sems = (sem0, sem1)

    _zero_vmem_2d(zbuf, ZR)
    r0 = sid * STRIPE

    @pl.when(sid < NS - 1)
    def _():
        for r in range(STRIPE // ZR):
            pltpu.sync_copy(zbuf, table_sh.at[pl.ds(r0 + r * ZR, ZR)])

    @pl.when(sid == NS - 1)
    def _():
        for r in range(LAST // ZR):
            pltpu.sync_copy(zbuf, table_sh.at[pl.ds(r0 + r * ZR, ZR)])
    plsc.subcore_barrier()

    for b in range(2):
        cp = wid + b * NW

        @pl.when(cp < NCH_E)
        def _(b=b, cp=cp):
            pltpu.sync_copy(src_hbm.at[pl.ds(cp * CH, CH)], sbufs[b].at[0])
            pltpu.async_copy(y_hbm.at[sbufs[b].at[0]], rbufs[b], sems[b])

    def outer(g, _):
        for b in range(2):
            k = 2 * g + b
            c = wid + k * NW

            @pl.when(c < NCH_E)
            def _(b=b, c=c):
                pltpu.make_async_copy(
                    y_hbm.at[pl.ds(0, CH)], rbufs[b], sems[b]).wait()
                pltpu.sync_copy(dst_hbm.at[pl.ds(c * CH, CH)], didx.at[0])
                pltpu.sync_copy(rbufs[b], table_sh.at[didx.at[0]], add=True)

                @pl.when(c + 2 * NW < NCH_E)
                def _():
                    base2 = (c + 2 * NW) * CH
                    pltpu.sync_copy(src_hbm.at[pl.ds(base2, CH)],
                                    sbufs[b].at[0])
                    pltpu.async_copy(y_hbm.at[sbufs[b].at[0]], rbufs[b],
                                     sems[b])
        return 0

    lax.fori_loop(0, (EK + 1) // 2, outer, 0)
    plsc.subcore_barrier()

    @pl.when(sid < NS - 1)
    def _():
        pltpu.sync_copy(table_sh.at[pl.ds(r0, STRIPE)],
                        pout.at[pl.ds(cid * N + r0, STRIPE)])

    @pl.when(sid == NS - 1)
    def _():
        pltpu.sync_copy(table_sh.at[pl.ds(r0, LAST)],
                        pout.at[pl.ds(cid * N + r0, LAST)])


_segment_sum = functools.partial(
    pl.kernel, _seg_body,
    out_type=jax.ShapeDtypeStruct((NC * N, H), jnp.float32),
    mesh=_mesh,
    scratch_types=[
        pltpu.VMEM((1, CH), jnp.int32),
        pltpu.VMEM((1, CH), jnp.int32),
        pltpu.VMEM((1, CH), jnp.int32),
        pltpu.VMEM((CH, H), jnp.float32),
        pltpu.VMEM((CH, H), jnp.float32),
        pltpu.VMEM((ZR, H), jnp.float32),
        pltpu.SemaphoreType.DMA,
        pltpu.SemaphoreType.DMA,
        pltpu.VMEM_SHARED((N, H), jnp.float32),
    ],
)()


def _cnt_body(dst_hbm, ones_hbm, cntout, didx, ones_v, zbuf, cnt_sh):
    cid = lax.axis_index("c")
    sid = lax.axis_index("s")
    wid = sid * NC + cid

    pltpu.sync_copy(ones_hbm, ones_v)
    _zero_vmem_2d(zbuf, ZR)
    r0 = sid * STRIPE

    @pl.when(sid < NS - 1)
    def _():
        for r in range(STRIPE // ZR):
            pltpu.sync_copy(zbuf, cnt_sh.at[pl.ds(r0 + r * ZR, ZR)])

    @pl.when(sid == NS - 1)
    def _():
        for r in range(LAST // ZR):
            pltpu.sync_copy(zbuf, cnt_sh.at[pl.ds(r0 + r * ZR, ZR)])
    plsc.subcore_barrier()

    def chunk(k, _):
        c = wid + k * NW

        @pl.when(c < NCH_E)
        def _():
            base = c * CH
            pltpu.sync_copy(dst_hbm.at[pl.ds(base, CH)], didx.at[0])
            pltpu.sync_copy(ones_v, cnt_sh.at[didx.at[0]], add=True)
        return 0

    lax.fori_loop(0, EK, chunk, 0)
    plsc.subcore_barrier()

    @pl.when(sid < NS - 1)
    def _():
        pltpu.sync_copy(cnt_sh.at[pl.ds(r0, STRIPE)],
                        cntout.at[pl.ds(cid * N + r0, STRIPE)])

    @pl.when(sid == NS - 1)
    def _():
        pltpu.sync_copy(cnt_sh.at[pl.ds(r0, LAST)],
                        cntout.at[pl.ds(cid * N + r0, LAST)])


_count_dst = functools.partial(
    pl.kernel, _cnt_body,
    out_type=jax.ShapeDtypeStruct((NC * N, H), jnp.float32),
    mesh=_mesh,
    scratch_types=[
        pltpu.VMEM((1, CH), jnp.int32),
        pltpu.VMEM((CH, H), jnp.float32),
        pltpu.VMEM((ZR, H), jnp.float32),
        pltpu.VMEM_SHARED((N, H), jnp.float32),
    ],
)()


def _pair_body(zu_hbm, zv_hbm, u_hbm, v_hbm, g1out, g2out,
               uidx, vidx, bufa, sem):
    cid = lax.axis_index("c")
    sid = lax.axis_index("s")
    wid = sid * NC + cid

    def chunk(k, _):
        c = wid + k * NW

        @pl.when(c < NCH_P)
        def _():
            base = jnp.minimum(c * CH, P - CH)
            pltpu.sync_copy(u_hbm.at[pl.ds(base, CH)], uidx.at[0])
            pltpu.sync_copy(v_hbm.at[pl.ds(base, CH)], vidx.at[0])
            pltpu.async_copy(zu_hbm.at[uidx.at[0]], bufa, sem).wait()
            pltpu.sync_copy(bufa, g1out.at[pl.ds(base, CH)])
            pltpu.async_copy(zv_hbm.at[vidx.at[0]], bufa, sem).wait()
            pltpu.sync_copy(bufa, g2out.at[pl.ds(base, CH)])
        return 0

    lax.fori_loop(0, PK, chunk, 0)


_pair_gather = functools.partial(
    pl.kernel, _pair_body,
    out_type=[jax.ShapeDtypeStruct((P, H), jnp.float32),
              jax.ShapeDtypeStruct((P, H), jnp.float32)],
    mesh=_mesh,
    scratch_types=[
        pltpu.VMEM((1, CH), jnp.int32),
        pltpu.VMEM((1, CH), jnp.int32),
        pltpu.VMEM((CH, H), jnp.float32),
        pltpu.SemaphoreType.DMA,
    ],
)()


BN = 1000
BP = 1000


def _pre_body(x_ref, wa_ref, wb_ref, ya_ref, yb_ref):
    xb = x_ref[...]
    ya_ref[...] = jnp.dot(xb, wa_ref[...], preferred_element_type=jnp.float32)
    yb_ref[...] = jnp.dot(xb, wb_ref[...], preferred_element_type=jnp.float32)


_pre_transform = pl.pallas_call(
    _pre_body,
    grid=(N // BN,),
    in_specs=[
        pl.BlockSpec((BN, D), lambda i: (i, 0)),
        pl.BlockSpec((D, H), lambda i: (0, 0)),
        pl.BlockSpec((D, H), lambda i: (0, 0)),
    ],
    out_specs=[
        pl.BlockSpec((BN, H), lambda i: (i, 0)),
        pl.BlockSpec((BN, H), lambda i: (i, 0)),
    ],
    out_shape=[jax.ShapeDtypeStruct((N, H), jnp.float32),
               jax.ShapeDtypeStruct((N, H), jnp.float32)],
)


def _comb_body(p0_ref, p1_ref, c0_ref, c1_ref, yr_ref, bl_ref,
               wa_ref, wb_ref, oa_ref, ob_ref):
    cnt = c0_ref[:, :1] + c1_ref[:, :1]
    recip = 1.0 / jnp.maximum(cnt, 1.0)
    z = (p0_ref[...] + p1_ref[...]) * recip + bl_ref[...] + yr_ref[...]
    z = jnp.maximum(z, 0.0)
    oa_ref[...] = jnp.dot(z, wa_ref[...], preferred_element_type=jnp.float32)
    ob_ref[...] = jnp.dot(z, wb_ref[...], preferred_element_type=jnp.float32)


_combine = pl.pallas_call(
    _comb_body,
    grid=(N // BN,),
    in_specs=[
        pl.BlockSpec((BN, H), lambda i: (i, 0)),
        pl.BlockSpec((BN, H), lambda i: (N // BN + i, 0)),
        pl.BlockSpec((BN, H), lambda i: (i, 0)),
        pl.BlockSpec((BN, H), lambda i: (N // BN + i, 0)),
        pl.BlockSpec((BN, H), lambda i: (i, 0)),
        pl.BlockSpec((1, H), lambda i: (0, 0)),
        pl.BlockSpec((H, H), lambda i: (0, 0)),
        pl.BlockSpec((H, H), lambda i: (0, 0)),
    ],
    out_specs=[
        pl.BlockSpec((BN, H), lambda i: (i, 0)),
        pl.BlockSpec((BN, H), lambda i: (i, 0)),
    ],
    out_shape=[jax.ShapeDtypeStruct((N, H), jnp.float32),
               jax.ShapeDtypeStruct((N, H), jnp.float32)],
)


def _dec_body(g1_ref, g2_ref, ea_ref, w1e_ref, b1_ref, w2_ref, b2_ref, out_ref):
    t = (g1_ref[...] + g2_ref[...] + b1_ref[...]
         + jnp.dot(ea_ref[...], w1e_ref[...], preferred_element_type=jnp.float32))
    t = jnp.maximum(t, 0.0)
    out_ref[0, 0, :] = jnp.sum(t * w2_ref[...], axis=1) + b2_ref[0, 0]


_decode = pl.pallas_call(
    _dec_body,
    grid=(P // BP,),
    in_specs=[
        pl.BlockSpec((BP, H), lambda i: (i, 0)),
        pl.BlockSpec((BP, H), lambda i: (i, 0)),
        pl.BlockSpec((BP, D), lambda i: (i, 0)),
        pl.BlockSpec((D, H), lambda i: (0, 0)),
        pl.BlockSpec((1, H), lambda i: (0, 0)),
        pl.BlockSpec((1, H), lambda i: (0, 0)),
        pl.BlockSpec((1, 1), lambda i: (0, 0)),
    ],
    out_specs=pl.BlockSpec((1, 1, BP), lambda i: (i, 0, 0)),
    out_shape=jax.ShapeDtypeStruct((P // BP, 1, BP), jnp.float32),
)


def kernel(x, edge_index, edge_pairs, edge_attr, Wl1, bl1, Wr1,
           Wl2, bl2, Wr2, W1, b1, W2, b2):
    src = edge_index[0]
    dst = edge_index[1]
    u = edge_pairs[0]
    v = edge_pairs[1]

    W1uT = W1[:, :H].T
    W1vT = W1[:, H:2 * H].T
    W1eT = W1[:, 2 * H:].T
    b1r = b1.reshape(1, H)
    w2r = W2.reshape(1, H)
    b2r = b2.reshape(1, 1)

    yl1, yr1 = _pre_transform(x, Wl1.T, Wr1.T)
    cnt2 = _count_dst(dst, jnp.ones((CH, H), jnp.float32))
    p1 = _segment_sum(yl1, src, dst)
    yl2, yr2 = _combine(p1, p1, cnt2, cnt2, yr1, bl1.reshape(1, H),
                        Wl2.T, Wr2.T)
    p2 = _segment_sum(yl2, src, dst)
    zu, zv = _combine(p2, p2, cnt2, cnt2, yr2, bl2.reshape(1, H),
                      W1uT, W1vT)
    g1, g2 = _pair_gather(zu, zv, u, v)
    out = _decode(g1, g2, edge_attr, W1eT, b1r, w2r, b2r)
    return out.reshape(P)


# speedup vs baseline: 5.6615x; 1.0734x over previous
import functools

import jax
import jax.numpy as jnp
from jax import lax
from jax.experimental import pallas as pl
from jax.experimental.pallas import tpu as pltpu
from jax.experimental.pallas import tpu_sc as plsc

N = 10000
E = 320000
P = 100000
D = 128
H = 128

NC = 2
NS = 16
NW = NC * NS
CH = 128

STRIPE = 640
LAST = N - (NS - 1) * STRIPE
ZR = 8
NCH_E = E // CH
NCH_P = P // CH + 1
EK = (NCH_E + NW - 1) // NW
PK = (NCH_P + NW - 1) // NW

_mesh = plsc.VectorSubcoreMesh(core_axis_name="c", subcore_axis_name="s")


def _zero_vmem_2d(ref, nrows):
    def body(i, _):
        ref[i // 8, pl.ds((i % 8) * 16, 16)] = jnp.zeros((16,), jnp.float32)
        return 0
    lax.fori_loop(0, nrows * 8, body, 0)


def _seg_body(y_hbm, src_hbm, dst_hbm, pout,
              sidx0, sidx1, didx, rows0, rows1, zbuf, sem0, sem1, table_sh):
    cid = lax.axis_index("c")
    sid = lax.axis_index("s")
    wid = sid * NC + cid
    sbufs = (sidx0, sidx1)
    rbufs = (rows0, rows1)
    sems = (sem0, sem1)

    _zero_vmem_2d(zbuf, ZR)
    r0 = sid * STRIPE

    @pl.when(sid < NS - 1)
    def _():
        for r in range(STRIPE // ZR):
            pltpu.sync_copy(zbuf, table_sh.at[pl.ds(r0 + r * ZR, ZR)])

    @pl.when(sid == NS - 1)
    def _():
        for r in range(LAST // ZR):
            pltpu.sync_copy(zbuf, table_sh.at[pl.ds(r0 + r * ZR, ZR)])
    plsc.subcore_barrier()

    for b in range(2):
        cp = wid + b * NW

        @pl.when(cp < NCH_E)
        def _(b=b, cp=cp):
            pltpu.sync_copy(src_hbm.at[pl.ds(cp * CH, CH)], sbufs[b].at[0])
            pltpu.async_copy(y_hbm.at[sbufs[b].at[0]], rbufs[b], sems[b])

    def outer(g, _):
        for b in range(2):
            k = 2 * g + b
            c = wid + k * NW

            @pl.when(c < NCH_E)
            def _(b=b, c=c):
                pltpu.make_async_copy(
                    y_hbm.at[pl.ds(0, CH)], rbufs[b], sems[b]).wait()
                pltpu.sync_copy(dst_hbm.at[pl.ds(c * CH, CH)], didx.at[0])
                pltpu.sync_copy(rbufs[b], table_sh.at[didx.at[0]], add=True)

                @pl.when(c + 2 * NW < NCH_E)
                def _():
                    base2 = (c + 2 * NW) * CH
                    pltpu.sync_copy(src_hbm.at[pl.ds(base2, CH)],
                                    sbufs[b].at[0])
                    pltpu.async_copy(y_hbm.at[sbufs[b].at[0]], rbufs[b],
                                     sems[b])
        return 0

    lax.fori_loop(0, (EK + 1) // 2, outer, 0)
    plsc.subcore_barrier()

    @pl.when(sid < NS - 1)
    def _():
        pltpu.sync_copy(table_sh.at[pl.ds(r0, STRIPE)],
                        pout.at[pl.ds(cid * N + r0, STRIPE)])

    @pl.when(sid == NS - 1)
    def _():
        pltpu.sync_copy(table_sh.at[pl.ds(r0, LAST)],
                        pout.at[pl.ds(cid * N + r0, LAST)])


_segment_sum = functools.partial(
    pl.kernel, _seg_body,
    out_type=jax.ShapeDtypeStruct((NC * N, H), jnp.float32),
    mesh=_mesh,
    scratch_types=[
        pltpu.VMEM((1, CH), jnp.int32),
        pltpu.VMEM((1, CH), jnp.int32),
        pltpu.VMEM((1, CH), jnp.int32),
        pltpu.VMEM((CH, H), jnp.float32),
        pltpu.VMEM((CH, H), jnp.float32),
        pltpu.VMEM((ZR, H), jnp.float32),
        pltpu.SemaphoreType.DMA,
        pltpu.SemaphoreType.DMA,
        pltpu.VMEM_SHARED((N, H), jnp.float32),
    ],
)()


def _cnt_body(dst_hbm, ones_hbm, cntout, didx, ones_v, zbuf, cnt_sh):
    cid = lax.axis_index("c")
    sid = lax.axis_index("s")
    wid = sid * NC + cid

    pltpu.sync_copy(ones_hbm, ones_v)
    _zero_vmem_2d(zbuf, ZR)
    r0 = sid * STRIPE

    @pl.when(sid < NS - 1)
    def _():
        for r in range(STRIPE // ZR):
            pltpu.sync_copy(zbuf, cnt_sh.at[pl.ds(r0 + r * ZR, ZR)])

    @pl.when(sid == NS - 1)
    def _():
        for r in range(LAST // ZR):
            pltpu.sync_copy(zbuf, cnt_sh.at[pl.ds(r0 + r * ZR, ZR)])
    plsc.subcore_barrier()

    def chunk(k, _):
        c = wid + k * NW

        @pl.when(c < NCH_E)
        def _():
            base = c * CH
            pltpu.sync_copy(dst_hbm.at[pl.ds(base, CH)], didx.at[0])
            pltpu.sync_copy(ones_v, cnt_sh.at[didx.at[0]], add=True)
        return 0

    lax.fori_loop(0, EK, chunk, 0)
    plsc.subcore_barrier()

    @pl.when(sid < NS - 1)
    def _():
        pltpu.sync_copy(cnt_sh.at[pl.ds(r0, STRIPE)],
                        cntout.at[pl.ds(cid * N + r0, STRIPE)])

    @pl.when(sid == NS - 1)
    def _():
        pltpu.sync_copy(cnt_sh.at[pl.ds(r0, LAST)],
                        cntout.at[pl.ds(cid * N + r0, LAST)])


_count_dst = functools.partial(
    pl.kernel, _cnt_body,
    out_type=jax.ShapeDtypeStruct((NC * N, H), jnp.float32),
    mesh=_mesh,
    scratch_types=[
        pltpu.VMEM((1, CH), jnp.int32),
        pltpu.VMEM((CH, H), jnp.float32),
        pltpu.VMEM((ZR, H), jnp.float32),
        pltpu.VMEM_SHARED((N, H), jnp.float32),
    ],
)()


def _pair_body(zu_hbm, zv_hbm, u_hbm, v_hbm, g1out, g2out,
               uidx0, vidx0, uidx1, vidx1, bufa0, bufb0, bufa1, bufb1,
               sem0, sem1):
    cid = lax.axis_index("c")
    sid = lax.axis_index("s")
    wid = sid * NC + cid
    ubufs = (uidx0, uidx1)
    vbufs = (vidx0, vidx1)
    abufs = (bufa0, bufa1)
    bbufs = (bufb0, bufb1)
    sems = (sem0, sem1)

    def issue(c, b):
        base = jnp.minimum(c * CH, P - CH)
        pltpu.sync_copy(u_hbm.at[pl.ds(base, CH)], ubufs[b].at[0])
        pltpu.sync_copy(v_hbm.at[pl.ds(base, CH)], vbufs[b].at[0])
        pltpu.async_copy(zu_hbm.at[ubufs[b].at[0]], abufs[b], sems[b])
        pltpu.async_copy(zv_hbm.at[vbufs[b].at[0]], bbufs[b], sems[b])

    for b in range(2):
        cp = wid + b * NW

        @pl.when(cp < NCH_P)
        def _(b=b, cp=cp):
            issue(cp, b)

    def outer(g, _):
        for b in range(2):
            k = 2 * g + b
            c = wid + k * NW

            @pl.when(c < NCH_P)
            def _(b=b, c=c):
                pltpu.make_async_copy(
                    zu_hbm.at[pl.ds(0, CH)], abufs[b], sems[b]).wait()
                pltpu.make_async_copy(
                    zu_hbm.at[pl.ds(0, CH)], bbufs[b], sems[b]).wait()
                base = jnp.minimum(c * CH, P - CH)
                pltpu.sync_copy(abufs[b], g1out.at[pl.ds(base, CH)])
                pltpu.sync_copy(bbufs[b], g2out.at[pl.ds(base, CH)])

                @pl.when(c + 2 * NW < NCH_P)
                def _():
                    issue(c + 2 * NW, b)
        return 0

    lax.fori_loop(0, (PK + 1) // 2, outer, 0)


_pair_gather = functools.partial(
    pl.kernel, _pair_body,
    out_type=[jax.ShapeDtypeStruct((P, H), jnp.float32),
              jax.ShapeDtypeStruct((P, H), jnp.float32)],
    mesh=_mesh,
    scratch_types=[
        pltpu.VMEM((1, CH), jnp.int32),
        pltpu.VMEM((1, CH), jnp.int32),
        pltpu.VMEM((1, CH), jnp.int32),
        pltpu.VMEM((1, CH), jnp.int32),
        pltpu.VMEM((CH, H), jnp.float32),
        pltpu.VMEM((CH, H), jnp.float32),
        pltpu.VMEM((CH, H), jnp.float32),
        pltpu.VMEM((CH, H), jnp.float32),
        pltpu.SemaphoreType.DMA,
        pltpu.SemaphoreType.DMA,
    ],
)()


BN = 1000
BP = 1000


def _pre_body(x_ref, wa_ref, wb_ref, ya_ref, yb_ref):
    xb = x_ref[...]
    ya_ref[...] = jnp.dot(xb, wa_ref[...], preferred_element_type=jnp.float32)
    yb_ref[...] = jnp.dot(xb, wb_ref[...], preferred_element_type=jnp.float32)


_pre_transform = pl.pallas_call(
    _pre_body,
    grid=(N // BN,),
    in_specs=[
        pl.BlockSpec((BN, D), lambda i: (i, 0)),
        pl.BlockSpec((D, H), lambda i: (0, 0)),
        pl.BlockSpec((D, H), lambda i: (0, 0)),
    ],
    out_specs=[
        pl.BlockSpec((BN, H), lambda i: (i, 0)),
        pl.BlockSpec((BN, H), lambda i: (i, 0)),
    ],
    out_shape=[jax.ShapeDtypeStruct((N, H), jnp.float32),
               jax.ShapeDtypeStruct((N, H), jnp.float32)],
)


def _comb_body(p0_ref, p1_ref, c0_ref, c1_ref, yr_ref, bl_ref,
               wa_ref, wb_ref, oa_ref, ob_ref):
    cnt = c0_ref[:, :1] + c1_ref[:, :1]
    recip = 1.0 / jnp.maximum(cnt, 1.0)
    z = (p0_ref[...] + p1_ref[...]) * recip + bl_ref[...] + yr_ref[...]
    z = jnp.maximum(z, 0.0)
    oa_ref[...] = jnp.dot(z, wa_ref[...], preferred_element_type=jnp.float32)
    ob_ref[...] = jnp.dot(z, wb_ref[...], preferred_element_type=jnp.float32)


_combine = pl.pallas_call(
    _comb_body,
    grid=(N // BN,),
    in_specs=[
        pl.BlockSpec((BN, H), lambda i: (i, 0)),
        pl.BlockSpec((BN, H), lambda i: (N // BN + i, 0)),
        pl.BlockSpec((BN, H), lambda i: (i, 0)),
        pl.BlockSpec((BN, H), lambda i: (N // BN + i, 0)),
        pl.BlockSpec((BN, H), lambda i: (i, 0)),
        pl.BlockSpec((1, H), lambda i: (0, 0)),
        pl.BlockSpec((H, H), lambda i: (0, 0)),
        pl.BlockSpec((H, H), lambda i: (0, 0)),
    ],
    out_specs=[
        pl.BlockSpec((BN, H), lambda i: (i, 0)),
        pl.BlockSpec((BN, H), lambda i: (i, 0)),
    ],
    out_shape=[jax.ShapeDtypeStruct((N, H), jnp.float32),
               jax.ShapeDtypeStruct((N, H), jnp.float32)],
)


def _dec_body(g1_ref, g2_ref, ea_ref, w1e_ref, b1_ref, w2_ref, b2_ref, out_ref):
    t = (g1_ref[...] + g2_ref[...] + b1_ref[...]
         + jnp.dot(ea_ref[...], w1e_ref[...], preferred_element_type=jnp.float32))
    t = jnp.maximum(t, 0.0)
    out_ref[0, 0, :] = jnp.sum(t * w2_ref[...], axis=1) + b2_ref[0, 0]


_decode = pl.pallas_call(
    _dec_body,
    grid=(P // BP,),
    in_specs=[
        pl.BlockSpec((BP, H), lambda i: (i, 0)),
        pl.BlockSpec((BP, H), lambda i: (i, 0)),
        pl.BlockSpec((BP, D), lambda i: (i, 0)),
        pl.BlockSpec((D, H), lambda i: (0, 0)),
        pl.BlockSpec((1, H), lambda i: (0, 0)),
        pl.BlockSpec((1, H), lambda i: (0, 0)),
        pl.BlockSpec((1, 1), lambda i: (0, 0)),
    ],
    out_specs=pl.BlockSpec((1, 1, BP), lambda i: (i, 0, 0)),
    out_shape=jax.ShapeDtypeStruct((P // BP, 1, BP), jnp.float32),
)


def kernel(x, edge_index, edge_pairs, edge_attr, Wl1, bl1, Wr1,
           Wl2, bl2, Wr2, W1, b1, W2, b2):
    src = edge_index[0]
    dst = edge_index[1]
    u = edge_pairs[0]
    v = edge_pairs[1]

    W1uT = W1[:, :H].T
    W1vT = W1[:, H:2 * H].T
    W1eT = W1[:, 2 * H:].T
    b1r = b1.reshape(1, H)
    w2r = W2.reshape(1, H)
    b2r = b2.reshape(1, 1)

    yl1, yr1 = _pre_transform(x, Wl1.T, Wr1.T)
    cnt2 = _count_dst(dst, jnp.ones((CH, H), jnp.float32))
    p1 = _segment_sum(yl1, src, dst)
    yl2, yr2 = _combine(p1, p1, cnt2, cnt2, yr1, bl1.reshape(1, H),
                        Wl2.T, Wr2.T)
    p2 = _segment_sum(yl2, src, dst)
    zu, zv = _combine(p2, p2, cnt2, cnt2, yr2, bl2.reshape(1, H),
                      W1uT, W1vT)
    g1, g2 = _pair_gather(zu, zv, u, v)
    out = _decode(g1, g2, edge_attr, W1eT, b1r, w2r, b2r)
    return out.reshape(P)
